# baseline (speedup 1.0000x reference)
import functools

import jax
import jax.numpy as jnp
from jax.experimental import pallas as pl
from jax.experimental.pallas import tpu as pltpu

_K = 1.0
_ALPHA = 0.25
_GAMMA = 2.0
_IGNORE_INDEX = -100

_ROWS = 1024
_COLS = 100000
_RBLK = 256
_CBLK = 2048
_NCBLK = (_COLS + _CBLK - 1) // _CBLK


def _focal_kernel(x_ref, tgt_ref, out_ref, m_ref, s_ref, t_ref):
    i = pl.program_id(0)
    j = pl.program_id(1)

    @pl.when(j == 0)
    def _init():
        m_ref[...] = jnp.full((_RBLK, 1), -jnp.inf, jnp.float32)
        s_ref[...] = jnp.zeros((_RBLK, 1), jnp.float32)
        t_ref[...] = jnp.zeros((_RBLK, 1), jnp.float32)

    x = x_ref[...]
    tgt = tgt_ref[...]
    col = j * _CBLK + jax.lax.broadcasted_iota(jnp.int32, (_RBLK, _CBLK), 1)

    t_ref[...] += jnp.sum(
        jnp.where(col == tgt, x, 0.0), axis=1, keepdims=True
    )

    is_last = j == _NCBLK - 1

    @pl.when(jnp.logical_not(is_last))
    def _full_block():
        m_old = m_ref[...]
        m_new = jnp.maximum(m_old, jnp.max(x, axis=1, keepdims=True))
        s_ref[...] = s_ref[...] * jnp.exp(m_old - m_new) + jnp.sum(
            jnp.exp(x - m_new), axis=1, keepdims=True
        )
        m_ref[...] = m_new

    @pl.when(is_last)
    def _last_block():
        xm = jnp.where(col < _COLS, x, -jnp.inf)
        m_old = m_ref[...]
        m_new = jnp.maximum(m_old, jnp.max(xm, axis=1, keepdims=True))
        s = s_ref[...] * jnp.exp(m_old - m_new) + jnp.sum(
            jnp.exp(xm - m_new), axis=1, keepdims=True
        )
        nll = m_new + jnp.log(s) - t_ref[...]
        loss = jnp.where(tgt == _IGNORE_INDEX, 0.0, nll)
        pt = jnp.exp(-loss)
        fl = _ALPHA * (1.0 - pt) * (1.0 - pt) * loss
        partial = jnp.sum(fl) * (1.0 / _ROWS)

        @pl.when(i == 0)
        def _zero():
            out_ref[0, 0] = 0.0

        out_ref[0, 0] += partial


def kernel(input, target):
    tgt2d = target.astype(jnp.int32).reshape(_ROWS, 1)
    out = pl.pallas_call(
        _focal_kernel,
        grid=(_ROWS // _RBLK, _NCBLK),
        in_specs=[
            pl.BlockSpec((_RBLK, _CBLK), lambda i, j: (i, j)),
            pl.BlockSpec((_RBLK, 1), lambda i, j: (i, 0)),
        ],
        out_specs=pl.BlockSpec(
            (1, 1), lambda i, j: (0, 0), memory_space=pltpu.SMEM
        ),
        out_shape=jax.ShapeDtypeStruct((1, 1), jnp.float32),
        scratch_shapes=[
            pltpu.VMEM((_RBLK, 1), jnp.float32),
            pltpu.VMEM((_RBLK, 1), jnp.float32),
            pltpu.VMEM((_RBLK, 1), jnp.float32),
        ],
    )(input, tgt2d)
    return out[0, 0]
